# baseline (speedup 1.0000x reference)
import functools

import jax
import jax.numpy as jnp
from jax import lax
from jax.experimental import pallas as pl
from jax.experimental.pallas import tpu as pltpu
from jax.experimental.pallas import tpu_sc as plsc

N_NODES = 10000
N_EDGES = 160000
D_IN = 256
D_OUT = 256

NC = 2
NS = 16
H = D_OUT // NC
CH = 128
K = 80
E_PAD = NS * K * CH
N_PAD = 10112
ROWS_PER_TILE = N_PAD // NS


def _mm_body(x_ref, w_ref, o_ref):
    o_ref[...] = jnp.dot(x_ref[...], w_ref[...],
                         preferred_element_type=jnp.float32)


def _support_matmul(x, w):
    bm = 1000
    return pl.pallas_call(
        _mm_body,
        grid=(N_NODES // bm,),
        in_specs=[
            pl.BlockSpec((bm, D_IN), lambda i: (i, 0)),
            pl.BlockSpec((D_IN, D_OUT), lambda i: (0, 0)),
        ],
        out_specs=pl.BlockSpec((bm, D_OUT), lambda i: (i, 0)),
        out_shape=jax.ShapeDtypeStruct((N_NODES, D_OUT), jnp.float32),
    )(x, w)


def _sc_body(sup_hbm, src_hbm, dst_hbm, w_hbm, b_hbm, out_hbm,
             idx_v, dst_v, w_v, rows_a, b_v, acc, sem_g):
    c = lax.axis_index("c")
    s = lax.axis_index("s")

    pltpu.sync_copy(src_hbm.at[s], idx_v)
    pltpu.sync_copy(dst_hbm.at[s], dst_v)
    pltpu.sync_copy(w_hbm.at[s], w_v)
    pltpu.sync_copy(b_hbm.at[c], b_v)

    def idx_body(i, carry):
        for k in range(CH // 16):
            sl = pl.ds(k * 16, 16)
            idx_v[i, sl] = idx_v[i, sl] * 2 + c
        return carry
    lax.fori_loop(0, K, idx_body, 0)

    bv = [b_v[pl.ds(k * 16, 16)] for k in range(H // 16)]

    def binit_body(i, carry):
        for k in range(H // 16):
            rows_a[i, pl.ds(k * 16, 16)] = bv[k]
        return carry
    lax.fori_loop(0, CH, binit_body, 0)

    base = s * ROWS_PER_TILE
    off = 0
    for sz in (128, 128, 128, 128, 120):
        pltpu.sync_copy(rows_a.at[pl.ds(0, sz)],
                        acc.at[pl.ds(base + off, sz)])
        off += sz
    plsc.subcore_barrier()

    def chunk_body(j, carry):
        pltpu.async_copy(sup_hbm.at[idx_v.at[j]], rows_a, sem_g).wait()

        def scale_body(g, carry2):
            wv = w_v[j, pl.ds(g * 16, 16)]
            for e in range(16):
                w = wv[e]
                i = g * 16 + e
                for k in range(H // 16):
                    sl = pl.ds(k * 16, 16)
                    rows_a[i, sl] = rows_a[i, sl] * w
            return carry2
        lax.fori_loop(0, CH // 16, scale_body, 0)

        pltpu.sync_copy(rows_a, acc.at[dst_v.at[j]], add=True)
        return carry
    lax.fori_loop(0, K, chunk_body, 0)

    plsc.subcore_barrier()

    pltpu.sync_copy(acc.at[pl.ds(base, ROWS_PER_TILE)],
                    out_hbm.at[c, pl.ds(base, ROWS_PER_TILE)])


def _sc_scatter(sup_flat, src3, dst3, w3, b2):
    mesh = plsc.VectorSubcoreMesh(core_axis_name="c", subcore_axis_name="s")
    f = functools.partial(
        pl.kernel,
        out_type=jax.ShapeDtypeStruct((NC, N_PAD, H), jnp.float32),
        mesh=mesh,
        scratch_types=[
            pltpu.VMEM((K, CH), jnp.int32),
            pltpu.VMEM((K, CH), jnp.int32),
            pltpu.VMEM((K, CH), jnp.float32),
            pltpu.VMEM((CH, H), jnp.float32),
            pltpu.VMEM((H,), jnp.float32),
            pltpu.VMEM_SHARED((N_PAD, H), jnp.float32),
            pltpu.SemaphoreType.DMA,
        ],
    )(_sc_body)
    return f(sup_flat, src3, dst3, w3, b2)


def kernel(edge_index, edge_weight, in_feature, W, b):
    support = _support_matmul(in_feature, W)
    sup_flat = support.reshape(NC * N_NODES, H)

    pad = E_PAD - N_EDGES
    src = jnp.concatenate([edge_index[1], jnp.zeros((pad,), jnp.int32)])
    dst = jnp.concatenate([edge_index[0], jnp.zeros((pad,), jnp.int32)])
    w = jnp.concatenate([edge_weight, jnp.zeros((pad,), jnp.float32)])
    src3 = src.reshape(NS, K, CH)
    dst3 = dst.reshape(NS, K, CH)
    w3 = w.reshape(NS, K, CH)
    b2 = b.reshape(NC, H)

    out = _sc_scatter(sup_flat, src3, dst3, w3, b2)
    return out[:, :N_NODES].transpose(1, 0, 2).reshape(N_NODES, D_OUT)


# speedup vs baseline: 1.1725x; 1.1725x over previous
import functools

import jax
import jax.numpy as jnp
from jax import lax
from jax.experimental import pallas as pl
from jax.experimental.pallas import tpu as pltpu
from jax.experimental.pallas import tpu_sc as plsc

N_NODES = 10000
N_EDGES = 160000
D_IN = 256
D_OUT = 256

NC = 2
NS = 16
H = D_OUT // NC
CH = 128
K = 79
E_PAD = NS * K * CH
N_PAD = 10112
ROWS_PER_TILE = N_PAD // NS


def _mm_body(x_ref, w_ref, o_ref):
    o_ref[...] = jnp.dot(x_ref[...], w_ref[...],
                         preferred_element_type=jnp.float32)


def _support_matmul(x, w):
    bm = 1000
    return pl.pallas_call(
        _mm_body,
        grid=(N_NODES // bm,),
        in_specs=[
            pl.BlockSpec((bm, D_IN), lambda i: (i, 0)),
            pl.BlockSpec((D_IN, D_OUT), lambda i: (0, 0)),
        ],
        out_specs=pl.BlockSpec((bm, D_OUT), lambda i: (i, 0)),
        out_shape=jax.ShapeDtypeStruct((N_NODES, D_OUT), jnp.float32),
    )(x, w)


def _sc_body(sup_hbm, src_hbm, dst_hbm, w_hbm, b_hbm, out_hbm,
             idx_v, dst_v, w_v, rows_a, b_v, acc, sem_g):
    c = lax.axis_index("c")
    s = lax.axis_index("s")

    pltpu.sync_copy(src_hbm.at[s], idx_v)
    pltpu.sync_copy(dst_hbm.at[s], dst_v)
    pltpu.sync_copy(w_hbm.at[s], w_v)
    pltpu.sync_copy(b_hbm.at[c], b_v)

    def idx_body(i, carry):
        for k in range(CH // 16):
            sl = pl.ds(k * 16, 16)
            idx_v[i, sl] = idx_v[i, sl] * 2 + c
        return carry
    lax.fori_loop(0, K, idx_body, 0)

    bv = [b_v[pl.ds(k * 16, 16)] for k in range(H // 16)]

    def binit_body(i, carry):
        for k in range(H // 16):
            rows_a[i, pl.ds(k * 16, 16)] = bv[k]
        return carry
    lax.fori_loop(0, CH, binit_body, 0)

    base = s * ROWS_PER_TILE
    off = 0
    for sz in (128, 128, 128, 128, 120):
        pltpu.sync_copy(rows_a.at[pl.ds(0, sz)],
                        acc.at[pl.ds(base + off, sz)])
        off += sz
    plsc.subcore_barrier()

    def chunk_body(j, carry):
        pltpu.async_copy(sup_hbm.at[idx_v.at[j]], rows_a, sem_g).wait()

        def scale_body(g, carry2):
            wv = w_v[j, pl.ds(g * 16, 16)]
            for e in range(16):
                w = wv[e]
                i = g * 16 + e
                for k in range(H // 16):
                    sl = pl.ds(k * 16, 16)
                    rows_a[i, sl] = rows_a[i, sl] * w
            return carry2
        lax.fori_loop(0, CH // 16, scale_body, 0)

        pltpu.sync_copy(rows_a, acc.at[dst_v.at[j]], add=True)
        return carry
    lax.fori_loop(0, K, chunk_body, 0)

    plsc.subcore_barrier()

    pltpu.sync_copy(acc.at[pl.ds(base, ROWS_PER_TILE)],
                    out_hbm.at[c, pl.ds(base, ROWS_PER_TILE)])


def _sc_scatter(sup_flat, src3, dst3, w3, b2):
    mesh = plsc.VectorSubcoreMesh(core_axis_name="c", subcore_axis_name="s")
    f = functools.partial(
        pl.kernel,
        out_type=jax.ShapeDtypeStruct((NC, N_PAD, H), jnp.float32),
        mesh=mesh,
        scratch_types=[
            pltpu.VMEM((K, CH), jnp.int32),
            pltpu.VMEM((K, CH), jnp.int32),
            pltpu.VMEM((K, CH), jnp.float32),
            pltpu.VMEM((CH, H), jnp.float32),
            pltpu.VMEM((H,), jnp.float32),
            pltpu.VMEM_SHARED((N_PAD, H), jnp.float32),
            pltpu.SemaphoreType.DMA,
        ],
    )(_sc_body)
    return f(sup_flat, src3, dst3, w3, b2)


def kernel(edge_index, edge_weight, in_feature, W, b):
    support = _support_matmul(in_feature, W)
    sup_flat = support.reshape(NC * N_NODES, H)

    pad = E_PAD - N_EDGES
    src = jnp.concatenate([edge_index[1], jnp.zeros((pad,), jnp.int32)])
    dst = jnp.concatenate([edge_index[0], jnp.zeros((pad,), jnp.int32)])
    w = jnp.concatenate([edge_weight, jnp.zeros((pad,), jnp.float32)])
    src3 = src.reshape(NS, K, CH)
    dst3 = dst.reshape(NS, K, CH)
    w3 = w.reshape(NS, K, CH)
    b2 = b.reshape(NC, H)

    out = _sc_scatter(sup_flat, src3, dst3, w3, b2)
    return out[:, :N_NODES].transpose(1, 0, 2).reshape(N_NODES, D_OUT)


# speedup vs baseline: 1.1734x; 1.0008x over previous
import functools

import jax
import jax.numpy as jnp
from jax import lax
from jax.experimental import pallas as pl
from jax.experimental.pallas import tpu as pltpu
from jax.experimental.pallas import tpu_sc as plsc

N_NODES = 10000
N_EDGES = 160000
D_IN = 256
D_OUT = 256

NC = 2
NS = 16
H = D_OUT // NC
CH = 128
K = 79
E_PAD = NS * K * CH
N_PAD = 10112
ROWS_PER_TILE = N_PAD // NS


def _mm_body(x_ref, w_ref, o_ref):
    o_ref[...] = jnp.dot(x_ref[...], w_ref[...],
                         preferred_element_type=jnp.float32)


def _support_matmul(x, w):
    bm = 1000
    return pl.pallas_call(
        _mm_body,
        grid=(N_NODES // bm,),
        in_specs=[
            pl.BlockSpec((bm, D_IN), lambda i: (i, 0)),
            pl.BlockSpec((D_IN, D_OUT), lambda i: (0, 0)),
        ],
        out_specs=pl.BlockSpec((bm, D_OUT), lambda i: (i, 0)),
        out_shape=jax.ShapeDtypeStruct((N_NODES, D_OUT), jnp.float32),
    )(x, w)


def _sc_body(sup_hbm, src_hbm, dst_hbm, w_hbm, b_hbm, out_hbm,
             idx_v, dst_v, w_v, rows_a, b_v, acc, sem_g):
    c = lax.axis_index("c")
    s = lax.axis_index("s")

    pltpu.sync_copy(src_hbm.at[s], idx_v)
    pltpu.sync_copy(dst_hbm.at[s], dst_v)
    pltpu.sync_copy(w_hbm.at[s], w_v)
    pltpu.sync_copy(b_hbm.at[c], b_v)

    def idx_body(i, carry):
        for k in range(CH // 16):
            sl = pl.ds(k * 16, 16)
            idx_v[i, sl] = idx_v[i, sl] * 2 + c
        return carry
    lax.fori_loop(0, K, idx_body, 0)

    bv = [b_v[pl.ds(k * 16, 16)] for k in range(H // 16)]

    def binit_body(i, carry):
        for k in range(H // 16):
            rows_a[i, pl.ds(k * 16, 16)] = bv[k]
        return carry
    lax.fori_loop(0, CH, binit_body, 0)

    base = s * ROWS_PER_TILE
    off = 0
    for sz in (128, 128, 128, 128, 120):
        pltpu.sync_copy(rows_a.at[pl.ds(0, sz)],
                        acc.at[pl.ds(base + off, sz)])
        off += sz
    plsc.subcore_barrier()

    def chunk_body(j, carry):
        d1 = pltpu.async_copy(sup_hbm.at[idx_v.at[j, pl.ds(0, 64)]],
                              rows_a.at[pl.ds(0, 64)], sem_g)
        d2 = pltpu.async_copy(sup_hbm.at[idx_v.at[j, pl.ds(64, 64)]],
                              rows_a.at[pl.ds(64, 64)], sem_g)
        d1.wait()
        d2.wait()

        def scale_body(g, carry2):
            wv = w_v[j, pl.ds(g * 16, 16)]
            for e in range(16):
                w = wv[e]
                i = g * 16 + e
                for k in range(H // 16):
                    sl = pl.ds(k * 16, 16)
                    rows_a[i, sl] = rows_a[i, sl] * w
            return carry2
        lax.fori_loop(0, CH // 16, scale_body, 0)

        pltpu.sync_copy(rows_a, acc.at[dst_v.at[j]], add=True)
        return carry
    lax.fori_loop(0, K, chunk_body, 0)

    plsc.subcore_barrier()

    pltpu.sync_copy(acc.at[pl.ds(base, ROWS_PER_TILE)],
                    out_hbm.at[c, pl.ds(base, ROWS_PER_TILE)])


def _sc_scatter(sup_flat, src3, dst3, w3, b2):
    mesh = plsc.VectorSubcoreMesh(core_axis_name="c", subcore_axis_name="s")
    f = functools.partial(
        pl.kernel,
        out_type=jax.ShapeDtypeStruct((NC, N_PAD, H), jnp.float32),
        mesh=mesh,
        scratch_types=[
            pltpu.VMEM((K, CH), jnp.int32),
            pltpu.VMEM((K, CH), jnp.int32),
            pltpu.VMEM((K, CH), jnp.float32),
            pltpu.VMEM((CH, H), jnp.float32),
            pltpu.VMEM((H,), jnp.float32),
            pltpu.VMEM_SHARED((N_PAD, H), jnp.float32),
            pltpu.SemaphoreType.DMA,
        ],
    )(_sc_body)
    return f(sup_flat, src3, dst3, w3, b2)


def kernel(edge_index, edge_weight, in_feature, W, b):
    support = _support_matmul(in_feature, W)
    sup_flat = support.reshape(NC * N_NODES, H)

    pad = E_PAD - N_EDGES
    src = jnp.concatenate([edge_index[1], jnp.zeros((pad,), jnp.int32)])
    dst = jnp.concatenate([edge_index[0], jnp.zeros((pad,), jnp.int32)])
    w = jnp.concatenate([edge_weight, jnp.zeros((pad,), jnp.float32)])
    src3 = src.reshape(NS, K, CH)
    dst3 = dst.reshape(NS, K, CH)
    w3 = w.reshape(NS, K, CH)
    b2 = b.reshape(NC, H)

    out = _sc_scatter(sup_flat, src3, dst3, w3, b2)
    return out[:, :N_NODES].transpose(1, 0, 2).reshape(N_NODES, D_OUT)


# speedup vs baseline: 1.2989x; 1.1069x over previous
import functools

import jax
import jax.numpy as jnp
from jax import lax
from jax.experimental import pallas as pl
from jax.experimental.pallas import tpu as pltpu
from jax.experimental.pallas import tpu_sc as plsc

N_NODES = 10000
N_EDGES = 160000
D_IN = 256
D_OUT = 256

NC = 2
NS = 16
H = D_OUT // NC
CH = 128
K = 79
E_PAD = NS * K * CH
N_PAD = 10112
ROWS_PER_TILE = N_PAD // NS


def _mm_body(x_ref, w_ref, o_ref):
    o_ref[...] = jnp.dot(x_ref[...], w_ref[...],
                         preferred_element_type=jnp.float32)


def _support_matmul(x, w):
    bm = 1000
    return pl.pallas_call(
        _mm_body,
        grid=(N_NODES // bm,),
        in_specs=[
            pl.BlockSpec((bm, D_IN), lambda i: (i, 0)),
            pl.BlockSpec((D_IN, D_OUT), lambda i: (0, 0)),
        ],
        out_specs=pl.BlockSpec((bm, D_OUT), lambda i: (i, 0)),
        out_shape=jax.ShapeDtypeStruct((N_NODES, D_OUT), jnp.float32),
    )(x, w)


def _sc_body(sup_hbm, src_hbm, dst_hbm, w_hbm, b_hbm, out_hbm,
             idx_v, dst_v, w_v, rows_a, b_v, acc, sem_g):
    c = lax.axis_index("c")
    s = lax.axis_index("s")

    pltpu.sync_copy(src_hbm.at[s], idx_v)
    pltpu.sync_copy(dst_hbm.at[s], dst_v)
    pltpu.sync_copy(w_hbm.at[s], w_v)
    pltpu.sync_copy(b_hbm.at[c], b_v)

    def idx_body(i, carry):
        for k in range(CH // 16):
            sl = pl.ds(k * 16, 16)
            idx_v[i, sl] = idx_v[i, sl] * 2 + c
        return carry
    lax.fori_loop(0, K, idx_body, 0)

    bv = [b_v[pl.ds(k * 16, 16)] for k in range(H // 16)]

    def binit_body(i, carry):
        for k in range(H // 16):
            rows_a[i, pl.ds(k * 16, 16)] = bv[k]
        return carry
    lax.fori_loop(0, CH, binit_body, 0)

    base = s * ROWS_PER_TILE
    off = 0
    for sz in (128, 128, 128, 128, 120):
        pltpu.sync_copy(rows_a.at[pl.ds(0, sz)],
                        acc.at[pl.ds(base + off, sz)])
        off += sz
    plsc.subcore_barrier()

    def chunk_body(j, carry):
        pltpu.async_copy(sup_hbm.at[idx_v.at[j]], rows_a, sem_g).wait()

        def scale_body(g, carry2):
            wv = w_v[j, pl.ds(g * 16, 16)]
            for e in range(16):
                w = wv[e]
                i = g * 16 + e
                for k in range(H // 16):
                    sl = pl.ds(k * 16, 16)
                    rows_a[i, sl] = rows_a[i, sl] * w
            return carry2
        lax.fori_loop(0, CH // 16, scale_body, 0)

        pltpu.sync_copy(rows_a, acc.at[dst_v.at[j]], add=True)
        return carry
    lax.fori_loop(0, K, chunk_body, 0)

    plsc.subcore_barrier()

    pltpu.sync_copy(acc.at[pl.ds(base, ROWS_PER_TILE)],
                    out_hbm.at[pl.ds(base, ROWS_PER_TILE), c])


def _sc_scatter(sup_flat, src3, dst3, w3, b2):
    mesh = plsc.VectorSubcoreMesh(core_axis_name="c", subcore_axis_name="s")
    f = functools.partial(
        pl.kernel,
        out_type=jax.ShapeDtypeStruct((N_PAD, NC, H), jnp.float32),
        mesh=mesh,
        scratch_types=[
            pltpu.VMEM((K, CH), jnp.int32),
            pltpu.VMEM((K, CH), jnp.int32),
            pltpu.VMEM((K, CH), jnp.float32),
            pltpu.VMEM((CH, H), jnp.float32),
            pltpu.VMEM((H,), jnp.float32),
            pltpu.VMEM_SHARED((N_PAD, H), jnp.float32),
            pltpu.SemaphoreType.DMA,
        ],
    )(_sc_body)
    return f(sup_flat, src3, dst3, w3, b2)


def kernel(edge_index, edge_weight, in_feature, W, b):
    support = _support_matmul(in_feature, W)
    sup_flat = support.reshape(NC * N_NODES, H)

    pad = E_PAD - N_EDGES
    src = jnp.concatenate([edge_index[1], jnp.zeros((pad,), jnp.int32)])
    dst = jnp.concatenate([edge_index[0], jnp.zeros((pad,), jnp.int32)])
    w = jnp.concatenate([edge_weight, jnp.zeros((pad,), jnp.float32)])
    src3 = src.reshape(NS, K, CH)
    dst3 = dst.reshape(NS, K, CH)
    w3 = w.reshape(NS, K, CH)
    b2 = b.reshape(NC, H)

    out = _sc_scatter(sup_flat, src3, dst3, w3, b2)
    return out.reshape(N_PAD, D_OUT)[:N_NODES]
